# Initial kernel scaffold; baseline (speedup 1.0000x reference)
import jax
import jax.numpy as jnp
from jax.experimental import pallas as pl


def kernel(node_embed, message_edge, W1, b1, W2, b2):
    raise NotImplementedError("write your pallas kernel here")


# speedup vs baseline: 13.0835x; 13.0835x over previous
import functools

import jax
import jax.numpy as jnp
from jax import lax
from jax.experimental import pallas as pl
from jax.experimental.pallas import tpu as pltpu
from jax.experimental.pallas import tpu_sc as plsc

NC = 2
NS = 16
LANES = 16

_MESH = plsc.VectorSubcoreMesh(
    core_axis_name="c", subcore_axis_name="s", num_cores=NC, num_subcores=NS
)


def _row_partition(n_nodes):
  rpt = (-(-n_nodes // NS) + 7) // 8 * 8
  return rpt, rpt * NS


def _deg_partials(n_nodes, n_edges):
  chunk = 80
  ept = n_edges // (NC * NS)
  n_chunks = ept // chunk
  rpt, npad = _row_partition(n_nodes)

  @functools.partial(
      pl.kernel,
      out_type=jax.ShapeDtypeStruct((NC, npad, LANES), jnp.float32),
      mesh=_MESH,
      scratch_types=[
          pltpu.VMEM_SHARED((npad, LANES), jnp.float32),
          pltpu.VMEM((rpt, LANES), jnp.float32),
          pltpu.VMEM((chunk, LANES), jnp.float32),
          pltpu.VMEM((chunk,), jnp.int32),
      ],
  )
  def k(dst_hbm, degp_hbm, acc_sh, zeros_v, ones_v, dst_v):
    cid = lax.axis_index("c")
    sid = lax.axis_index("s")

    @pl.loop(0, rpt)
    def _(r):
      zeros_v[r, :] = jnp.zeros((LANES,), jnp.float32)

    @pl.loop(0, chunk)
    def _(r):
      ones_v[r, :] = jnp.full((LANES,), 1.0, jnp.float32)

    pltpu.sync_copy(zeros_v, acc_sh.at[pl.ds(sid * rpt, rpt)])
    plsc.subcore_barrier()

    base = cid * (n_edges // NC) + sid * ept

    @pl.loop(0, n_chunks)
    def _(i):
      pltpu.sync_copy(dst_hbm.at[pl.ds(base + i * chunk, chunk)], dst_v)
      pltpu.sync_copy(ones_v, acc_sh.at[dst_v], add=True)

    plsc.subcore_barrier()
    pltpu.sync_copy(acc_sh.at[pl.ds(sid * rpt, rpt)],
                    degp_hbm.at[cid, pl.ds(sid * rpt, rpt)])

  return k


def _edge_scatter(n_nodes, n_edges, d):
  chunk = 80
  ept = n_edges // (NC * NS)
  n_chunks = ept // chunk
  rpt, npad = _row_partition(n_nodes)
  last_rows = n_nodes - (NS - 1) * rpt

  @functools.partial(
      pl.kernel,
      out_type=jax.ShapeDtypeStruct((NC, npad, d), jnp.float32),
      mesh=_MESH,
      scratch_types=[
          pltpu.VMEM_SHARED((npad, d), jnp.float32),
          pltpu.VMEM((chunk, d), jnp.float32),
          pltpu.VMEM((chunk,), jnp.int32),
          pltpu.VMEM((chunk,), jnp.int32),
          pltpu.SemaphoreType.DMA,
      ],
  )
  def k(h_hbm, src_hbm, dst_hbm, part_hbm, acc_sh, rows_v, src_v, dst_v, sem):
    cid = lax.axis_index("c")
    sid = lax.axis_index("s")

    @pl.when(sid < NS - 1)
    def _():
      pltpu.sync_copy(h_hbm.at[pl.ds(sid * rpt, rpt)],
                      acc_sh.at[pl.ds(sid * rpt, rpt)])

    @pl.when(sid == NS - 1)
    def _():
      pltpu.sync_copy(h_hbm.at[pl.ds((NS - 1) * rpt, last_rows)],
                      acc_sh.at[pl.ds((NS - 1) * rpt, last_rows)])

    plsc.subcore_barrier()

    base = cid * (n_edges // NC) + sid * ept

    @pl.loop(0, n_chunks)
    def _(i):
      off = base + i * chunk
      pltpu.sync_copy(src_hbm.at[pl.ds(off, chunk)], src_v)
      pltpu.sync_copy(dst_hbm.at[pl.ds(off, chunk)], dst_v)
      pltpu.async_copy(h_hbm.at[src_v], rows_v, sem).wait()
      pltpu.sync_copy(rows_v, acc_sh.at[dst_v], add=True)

    plsc.subcore_barrier()
    pltpu.sync_copy(acc_sh.at[pl.ds(sid * rpt, rpt)],
                    part_hbm.at[cid, pl.ds(sid * rpt, rpt)])

  return k


_ROWS = 2000


def _dis_from(degp_ref):
  deg = degp_ref[0, :, 0:1] + degp_ref[1, :, 0:1] + 1.0
  return lax.rsqrt(deg)


def _tc_matmul1(x_ref, w_ref, degp_ref, o_ref):
  dis = _dis_from(degp_ref)
  h = jnp.dot(x_ref[...], w_ref[...], preferred_element_type=jnp.float32)
  o_ref[...] = h * dis


def _tc_mid(part_ref, h_ref, degp_ref, w_ref, b_ref, o_ref):
  dis = _dis_from(degp_ref)
  s = part_ref[0] + part_ref[1] - h_ref[...]
  z = jnp.maximum(dis * s + b_ref[...], 0.0)
  h2 = jnp.dot(z, w_ref[...], preferred_element_type=jnp.float32)
  o_ref[...] = h2 * dis


def _tc_final(part_ref, h_ref, degp_ref, b_ref, o_ref):
  dis = _dis_from(degp_ref)
  s = part_ref[0] + part_ref[1] - h_ref[...]
  o_ref[...] = dis * s + b_ref[...]


def kernel(node_embed, message_edge, W1, b1, W2, b2):
  n, d_in = node_embed.shape
  d_hid = W1.shape[1]
  d_out = W2.shape[1]
  e = message_edge.shape[1]
  src = message_edge[0]
  dst = message_edge[1]
  b1r = b1.reshape(1, d_hid)
  b2r = b2.reshape(1, d_out)

  degp = _deg_partials(n, e)(dst)

  grid = (n // _ROWS,)
  blk_rows = lambda dd: pl.BlockSpec((_ROWS, dd), lambda i: (i, 0))
  blk_part = lambda dd: pl.BlockSpec((NC, _ROWS, dd), lambda i: (0, i, 0))
  blk_w = pl.BlockSpec((d_in, d_hid), lambda i: (0, 0))
  blk_b = lambda dd: pl.BlockSpec((1, dd), lambda i: (0, 0))

  h1p = pl.pallas_call(
      _tc_matmul1,
      grid=grid,
      in_specs=[blk_rows(d_in), blk_w, blk_part(LANES)],
      out_specs=blk_rows(d_hid),
      out_shape=jax.ShapeDtypeStruct((n, d_hid), jnp.float32),
  )(node_embed, W1, degp)

  scat = _edge_scatter(n, e, d_hid)
  part1 = scat(h1p, src, dst)

  h2p = pl.pallas_call(
      _tc_mid,
      grid=grid,
      in_specs=[blk_part(d_hid), blk_rows(d_hid), blk_part(LANES),
                pl.BlockSpec((d_hid, d_out), lambda i: (0, 0)), blk_b(d_hid)],
      out_specs=blk_rows(d_out),
      out_shape=jax.ShapeDtypeStruct((n, d_out), jnp.float32),
  )(part1, h1p, degp, W2, b1r)

  part2 = scat(h2p, src, dst)

  out = pl.pallas_call(
      _tc_final,
      grid=grid,
      in_specs=[blk_part(d_out), blk_rows(d_out), blk_part(LANES),
                blk_b(d_out)],
      out_specs=blk_rows(d_out),
      out_shape=jax.ShapeDtypeStruct((n, d_out), jnp.float32),
  )(part2, h2p, degp, b2r)

  return out


# speedup vs baseline: 30.9563x; 2.3661x over previous
import functools

import jax
import jax.numpy as jnp
from jax import lax
from jax.experimental import pallas as pl
from jax.experimental.pallas import tpu as pltpu
from jax.experimental.pallas import tpu_sc as plsc

NC = 2
NS = 16
NW = NC * NS
LANES = 16
CHUNK = 80
RPT = 632

_MESH = plsc.VectorSubcoreMesh(
    core_axis_name="c", subcore_axis_name="s", num_cores=NC, num_subcores=NS
)


def _deg_partials(n_nodes, n_chunks):
  last_rows = n_nodes - (NS - 1) * RPT
  ring = 8

  @functools.partial(
      pl.kernel,
      out_type=jax.ShapeDtypeStruct((NC, n_nodes, LANES), jnp.float32),
      mesh=_MESH,
      scratch_types=[
          pltpu.VMEM_SHARED((n_nodes, LANES), jnp.float32),
          pltpu.VMEM((RPT, LANES), jnp.float32),
          pltpu.VMEM((CHUNK, LANES), jnp.float32),
          pltpu.VMEM((n_chunks, CHUNK), jnp.int32),
          pltpu.SemaphoreType.DMA,
      ],
  )
  def k(dst_hbm, degp_hbm, acc_sh, zeros_v, ones_v, dsts_v, ssem):
    cid = lax.axis_index("c")
    sid = lax.axis_index("s")
    wid = cid * NS + sid

    @pl.loop(0, RPT)
    def _(r):
      zeros_v[r, :] = jnp.zeros((LANES,), jnp.float32)

    @pl.loop(0, CHUNK)
    def _(r):
      ones_v[r, :] = jnp.full((LANES,), 1.0, jnp.float32)

    pltpu.sync_copy(dst_hbm.at[wid], dsts_v)

    @pl.when(sid < NS - 1)
    def _():
      pltpu.sync_copy(zeros_v, acc_sh.at[pl.ds(sid * RPT, RPT)])

    @pl.when(sid == NS - 1)
    def _():
      pltpu.sync_copy(zeros_v.at[pl.ds(0, last_rows)],
                      acc_sh.at[pl.ds((NS - 1) * RPT, last_rows)])

    plsc.subcore_barrier()

    def drain():
      pltpu.make_async_copy(ones_v, acc_sh.at[dsts_v.at[0]], ssem).wait()

    @pl.loop(0, n_chunks)
    def _(i):
      @pl.when(i >= ring)
      def _():
        drain()
      pltpu.async_copy(ones_v, acc_sh.at[dsts_v.at[i]], ssem, add=True)

    for _ in range(min(ring, n_chunks)):
      drain()

    plsc.subcore_barrier()

    @pl.when(sid < NS - 1)
    def _():
      pltpu.sync_copy(acc_sh.at[pl.ds(sid * RPT, RPT)],
                      degp_hbm.at[cid, pl.ds(sid * RPT, RPT)])

    @pl.when(sid == NS - 1)
    def _():
      pltpu.sync_copy(acc_sh.at[pl.ds((NS - 1) * RPT, last_rows)],
                      degp_hbm.at[cid, pl.ds((NS - 1) * RPT, last_rows)])

  return k


def _edge_scatter(n_nodes, n_chunks, d):
  last_rows = n_nodes - (NS - 1) * RPT
  assert n_chunks % 4 == 1 and n_chunks >= 5

  @functools.partial(
      pl.kernel,
      out_type=jax.ShapeDtypeStruct((NC, n_nodes, d), jnp.float32),
      mesh=_MESH,
      scratch_types=[
          pltpu.VMEM_SHARED((n_nodes, d), jnp.float32),
          pltpu.VMEM((CHUNK, d), jnp.float32),
          pltpu.VMEM((CHUNK, d), jnp.float32),
          [pltpu.VMEM((CHUNK,), jnp.int32)] * 4,
          [pltpu.VMEM((CHUNK,), jnp.int32)] * 4,
          [pltpu.SemaphoreType.DMA] * 2,
          [pltpu.SemaphoreType.DMA] * 2,
          [pltpu.SemaphoreType.DMA] * 4,
      ],
  )
  def k(h_hbm, src_hbm, dst_hbm, part_hbm,
        acc_sh, rows0, rows1, sb, db, gsem, ssem, isem):
    cid = lax.axis_index("c")
    sid = lax.axis_index("s")
    wid = cid * NS + sid
    rows = (rows0, rows1)
    base = wid * (n_chunks * CHUNK)

    @pl.when(sid < NS - 1)
    def _():
      pltpu.sync_copy(h_hbm.at[pl.ds(sid * RPT, RPT)],
                      acc_sh.at[pl.ds(sid * RPT, RPT)])

    @pl.when(sid == NS - 1)
    def _():
      pltpu.sync_copy(h_hbm.at[pl.ds((NS - 1) * RPT, last_rows)],
                      acc_sh.at[pl.ds((NS - 1) * RPT, last_rows)])

    plsc.subcore_barrier()

    def li(j, q):
      off = base + j * CHUNK
      pltpu.async_copy(src_hbm.at[pl.ds(off, CHUNK)], sb[q], isem[q])
      pltpu.async_copy(dst_hbm.at[pl.ds(off, CHUNK)], db[q], isem[q])

    def wi(q):
      pltpu.make_async_copy(src_hbm.at[pl.ds(0, CHUNK)], sb[q], isem[q]).wait()
      pltpu.make_async_copy(dst_hbm.at[pl.ds(0, CHUNK)], db[q], isem[q]).wait()

    def g(b, q):
      pltpu.async_copy(h_hbm.at[sb[q]], rows[b], gsem[b])

    def wg(b):
      pltpu.make_async_copy(h_hbm.at[sb[0]], rows[b], gsem[b]).wait()

    def s(b, q):
      pltpu.async_copy(rows[b], acc_sh.at[db[q]], ssem[b], add=True)

    def ws(b):
      pltpu.make_async_copy(rows[0], acc_sh.at[db[0]], ssem[b]).wait()

    li(0, 0)
    li(1, 1)
    wi(0)
    g(0, 0)
    li(2, 2)
    wi(1)
    g(1, 1)
    li(3, 3)
    wg(0)
    s(0, 0)

    n_quads = (n_chunks - 7) // 4

    @pl.loop(0, n_quads)
    def _(i):
      j0 = 4 * i + 2
      for t in range(4):
        b, q = t % 2, (2 + t) % 4
        ws(b)
        li(j0 + t + 2, t)
        wi(q)
        g(b, q)
        wg(b ^ 1)
        s(b ^ 1, (1 + t) % 4)

    for j in range(4 * n_quads + 2, n_chunks):
      b, q = j % 2, j % 4
      ws(b)
      if j + 2 < n_chunks:
        li(j + 2, (j + 2) % 4)
      wi(q)
      g(b, q)
      wg(b ^ 1)
      s(b ^ 1, (j - 1) % 4)
    lastb = (n_chunks - 1) % 2
    wg(lastb)
    s(lastb, (n_chunks - 1) % 4)
    ws(0)
    ws(1)

    plsc.subcore_barrier()

    @pl.when(sid < NS - 1)
    def _():
      pltpu.sync_copy(acc_sh.at[pl.ds(sid * RPT, RPT)],
                      part_hbm.at[cid, pl.ds(sid * RPT, RPT)])

    @pl.when(sid == NS - 1)
    def _():
      pltpu.sync_copy(acc_sh.at[pl.ds((NS - 1) * RPT, last_rows)],
                      part_hbm.at[cid, pl.ds((NS - 1) * RPT, last_rows)])

  return k


_ROWS = 2000


def _dis_from(degp_ref):
  deg = degp_ref[0, :, 0:1] + degp_ref[1, :, 0:1] + 1.0
  return lax.rsqrt(deg)


def _tc_matmul1(x_ref, w_ref, degp_ref, o_ref):
  dis = _dis_from(degp_ref)
  h = jnp.dot(x_ref[...], w_ref[...], preferred_element_type=jnp.float32)
  o_ref[...] = h * dis


def _tc_mid(part_ref, h_ref, degp_ref, w_ref, b_ref, o_ref):
  dis = _dis_from(degp_ref)
  s = part_ref[0] + part_ref[1] - h_ref[...]
  z = jnp.maximum(dis * s + b_ref[...], 0.0)
  h2 = jnp.dot(z, w_ref[...], preferred_element_type=jnp.float32)
  o_ref[...] = h2 * dis


def _tc_final(part_ref, h_ref, degp_ref, b_ref, o_ref):
  dis = _dis_from(degp_ref)
  s = part_ref[0] + part_ref[1] - h_ref[...]
  o_ref[...] = dis * s + b_ref[...]


def kernel(node_embed, message_edge, W1, b1, W2, b2):
  n, d_in = node_embed.shape
  d_hid = W1.shape[1]
  d_out = W2.shape[1]
  e = message_edge.shape[1]
  ept = e // NW
  n_chunks = ept // CHUNK
  assert ept % CHUNK == 0 and e % NW == 0
  src = message_edge[0]
  dst = message_edge[1]
  dst3 = dst.reshape(NW, n_chunks, CHUNK)
  b1r = b1.reshape(1, d_hid)
  b2r = b2.reshape(1, d_out)

  degp = _deg_partials(n, n_chunks)(dst3)

  grid = (n // _ROWS,)
  blk_rows = lambda dd: pl.BlockSpec((_ROWS, dd), lambda i: (i, 0))
  blk_part = lambda dd: pl.BlockSpec((NC, _ROWS, dd), lambda i: (0, i, 0))
  blk_w = pl.BlockSpec((d_in, d_hid), lambda i: (0, 0))
  blk_b = lambda dd: pl.BlockSpec((1, dd), lambda i: (0, 0))

  h1p = pl.pallas_call(
      _tc_matmul1,
      grid=grid,
      in_specs=[blk_rows(d_in), blk_w, blk_part(LANES)],
      out_specs=blk_rows(d_hid),
      out_shape=jax.ShapeDtypeStruct((n, d_hid), jnp.float32),
  )(node_embed, W1, degp)

  scat = _edge_scatter(n, n_chunks, d_hid)
  part1 = scat(h1p, src, dst)

  h2p = pl.pallas_call(
      _tc_mid,
      grid=grid,
      in_specs=[blk_part(d_hid), blk_rows(d_hid), blk_part(LANES),
                pl.BlockSpec((d_hid, d_out), lambda i: (0, 0)), blk_b(d_hid)],
      out_specs=blk_rows(d_out),
      out_shape=jax.ShapeDtypeStruct((n, d_out), jnp.float32),
  )(part1, h1p, degp, W2, b1r)

  part2 = scat(h2p, src, dst)

  out = pl.pallas_call(
      _tc_final,
      grid=grid,
      in_specs=[blk_part(d_out), blk_rows(d_out), blk_part(LANES),
                blk_b(d_out)],
      out_specs=blk_rows(d_out),
      out_shape=jax.ShapeDtypeStruct((n, d_out), jnp.float32),
  )(part2, h2p, degp, b2r)

  return out


# speedup vs baseline: 34.3774x; 1.1105x over previous
import functools

import jax
import jax.numpy as jnp
from jax import lax
from jax.experimental import pallas as pl
from jax.experimental.pallas import tpu as pltpu
from jax.experimental.pallas import tpu_sc as plsc

NC = 2
NS = 16
NW = NC * NS
LANES = 16
CHUNK = 80
RPT = 632

_MESH = plsc.VectorSubcoreMesh(
    core_axis_name="c", subcore_axis_name="s", num_cores=NC, num_subcores=NS
)


def _deg_partials(n_nodes, n_chunks):
  last_rows = n_nodes - (NS - 1) * RPT
  ring = 8

  @functools.partial(
      pl.kernel,
      out_type=jax.ShapeDtypeStruct((NC, n_nodes, LANES), jnp.float32),
      mesh=_MESH,
      scratch_types=[
          pltpu.VMEM_SHARED((n_nodes, LANES), jnp.float32),
          pltpu.VMEM((RPT, LANES), jnp.float32),
          pltpu.VMEM((CHUNK, LANES), jnp.float32),
          pltpu.VMEM((n_chunks, CHUNK), jnp.int32),
          pltpu.SemaphoreType.DMA,
      ],
  )
  def k(dst_hbm, degp_hbm, acc_sh, zeros_v, ones_v, dsts_v, ssem):
    cid = lax.axis_index("c")
    sid = lax.axis_index("s")
    wid = cid * NS + sid

    @pl.loop(0, RPT)
    def _(r):
      zeros_v[r, :] = jnp.zeros((LANES,), jnp.float32)

    @pl.loop(0, CHUNK)
    def _(r):
      ones_v[r, :] = jnp.full((LANES,), 1.0, jnp.float32)

    pltpu.sync_copy(dst_hbm.at[wid], dsts_v)

    @pl.when(sid < NS - 1)
    def _():
      pltpu.sync_copy(zeros_v, acc_sh.at[pl.ds(sid * RPT, RPT)])

    @pl.when(sid == NS - 1)
    def _():
      pltpu.sync_copy(zeros_v.at[pl.ds(0, last_rows)],
                      acc_sh.at[pl.ds((NS - 1) * RPT, last_rows)])

    plsc.subcore_barrier()

    def drain():
      pltpu.make_async_copy(ones_v, acc_sh.at[dsts_v.at[0]], ssem).wait()

    @pl.loop(0, n_chunks)
    def _(i):
      @pl.when(i >= ring)
      def _():
        drain()
      pltpu.async_copy(ones_v, acc_sh.at[dsts_v.at[i]], ssem, add=True)

    for _ in range(min(ring, n_chunks)):
      drain()

    plsc.subcore_barrier()

    @pl.when(sid < NS - 1)
    def _():
      pltpu.sync_copy(acc_sh.at[pl.ds(sid * RPT, RPT)],
                      degp_hbm.at[cid, pl.ds(sid * RPT, RPT)])

    @pl.when(sid == NS - 1)
    def _():
      pltpu.sync_copy(acc_sh.at[pl.ds((NS - 1) * RPT, last_rows)],
                      degp_hbm.at[cid, pl.ds((NS - 1) * RPT, last_rows)])

  return k


def _edge_scatter(n_nodes, n_chunks, d):
  last_rows = n_nodes - (NS - 1) * RPT
  assert n_chunks >= 13

  @functools.partial(
      pl.kernel,
      out_type=jax.ShapeDtypeStruct((NC, n_nodes, d), jnp.float32),
      mesh=_MESH,
      scratch_types=[
          pltpu.VMEM_SHARED((n_nodes, d), jnp.float32),
          [pltpu.VMEM((CHUNK, d), jnp.float32)] * 4,
          [pltpu.VMEM((CHUNK,), jnp.int32)] * 8,
          [pltpu.VMEM((CHUNK,), jnp.int32)] * 8,
          [pltpu.SemaphoreType.DMA] * 4,
          [pltpu.SemaphoreType.DMA] * 4,
          [pltpu.SemaphoreType.DMA] * 8,
      ],
  )
  def k(h_hbm, src_hbm, dst_hbm, part_hbm,
        acc_sh, rows, sb, db, gsem, ssem, isem):
    cid = lax.axis_index("c")
    sid = lax.axis_index("s")
    wid = cid * NS + sid
    base = wid * (n_chunks * CHUNK)

    @pl.when(sid < NS - 1)
    def _():
      pltpu.sync_copy(h_hbm.at[pl.ds(sid * RPT, RPT)],
                      acc_sh.at[pl.ds(sid * RPT, RPT)])

    @pl.when(sid == NS - 1)
    def _():
      pltpu.sync_copy(h_hbm.at[pl.ds((NS - 1) * RPT, last_rows)],
                      acc_sh.at[pl.ds((NS - 1) * RPT, last_rows)])

    plsc.subcore_barrier()

    def li(j, q):
      off = base + j * CHUNK
      pltpu.async_copy(src_hbm.at[pl.ds(off, CHUNK)], sb[q], isem[q])
      pltpu.async_copy(dst_hbm.at[pl.ds(off, CHUNK)], db[q], isem[q])

    def wi(q):
      pltpu.make_async_copy(src_hbm.at[pl.ds(0, CHUNK)], sb[q], isem[q]).wait()
      pltpu.make_async_copy(dst_hbm.at[pl.ds(0, CHUNK)], db[q], isem[q]).wait()

    def g(b, q):
      pltpu.async_copy(h_hbm.at[sb[q]], rows[b], gsem[b])

    def wg(b):
      pltpu.make_async_copy(h_hbm.at[sb[0]], rows[b], gsem[b]).wait()

    def s(b, q):
      pltpu.async_copy(rows[b], acc_sh.at[db[q]], ssem[b], add=True)

    def ws(b):
      pltpu.make_async_copy(rows[0], acc_sh.at[db[0]], ssem[b]).wait()

    for q in range(8):
      li(q, q)
    wi(0)
    g(0, 0)
    wi(1)
    g(1, 1)
    wg(0)
    s(0, 0)
    wi(2)
    g(2, 2)
    wg(1)
    s(1, 1)
    wi(3)
    g(3, 3)

    n_oct = (n_chunks - 8) // 8

    @pl.loop(0, n_oct)
    def _(i):
      j0 = 8 * i + 4
      for t in range(8):
        b, q = t % 4, (4 + t) % 8
        ws(b)
        li(j0 + t + 4, t)
        wi(q)
        g(b, q)
        wg((t + 2) % 4)
        s((t + 2) % 4, (2 + t) % 8)

    for j in range(8 * n_oct + 4, n_chunks):
      b, q = j % 4, j % 8
      ws(b)
      if j + 4 < n_chunks:
        li(j + 4, (j + 4) % 8)
      wi(q)
      g(b, q)
      wg((j - 2) % 4)
      s((j - 2) % 4, (j - 2) % 8)
    for j in (n_chunks - 2, n_chunks - 1):
      wg(j % 4)
      s(j % 4, j % 8)
    for b in range(4):
      ws(b)

    plsc.subcore_barrier()

    @pl.when(sid < NS - 1)
    def _():
      pltpu.sync_copy(acc_sh.at[pl.ds(sid * RPT, RPT)],
                      part_hbm.at[cid, pl.ds(sid * RPT, RPT)])

    @pl.when(sid == NS - 1)
    def _():
      pltpu.sync_copy(acc_sh.at[pl.ds((NS - 1) * RPT, last_rows)],
                      part_hbm.at[cid, pl.ds((NS - 1) * RPT, last_rows)])

  return k


_ROWS = 2000


def _dis_from(degp_ref):
  deg = degp_ref[0, :, 0:1] + degp_ref[1, :, 0:1] + 1.0
  return lax.rsqrt(deg)


def _tc_matmul1(x_ref, w_ref, degp_ref, o_ref):
  dis = _dis_from(degp_ref)
  h = jnp.dot(x_ref[...], w_ref[...], preferred_element_type=jnp.float32)
  o_ref[...] = h * dis


def _tc_mid(part_ref, h_ref, degp_ref, w_ref, b_ref, o_ref):
  dis = _dis_from(degp_ref)
  s = part_ref[0] + part_ref[1] - h_ref[...]
  z = jnp.maximum(dis * s + b_ref[...], 0.0)
  h2 = jnp.dot(z, w_ref[...], preferred_element_type=jnp.float32)
  o_ref[...] = h2 * dis


def _tc_final(part_ref, h_ref, degp_ref, b_ref, o_ref):
  dis = _dis_from(degp_ref)
  s = part_ref[0] + part_ref[1] - h_ref[...]
  o_ref[...] = dis * s + b_ref[...]


def kernel(node_embed, message_edge, W1, b1, W2, b2):
  n, d_in = node_embed.shape
  d_hid = W1.shape[1]
  d_out = W2.shape[1]
  e = message_edge.shape[1]
  ept = e // NW
  n_chunks = ept // CHUNK
  assert ept % CHUNK == 0 and e % NW == 0
  src = message_edge[0]
  dst = message_edge[1]
  dst3 = dst.reshape(NW, n_chunks, CHUNK)
  b1r = b1.reshape(1, d_hid)
  b2r = b2.reshape(1, d_out)

  degp = _deg_partials(n, n_chunks)(dst3)

  grid = (n // _ROWS,)
  blk_rows = lambda dd: pl.BlockSpec((_ROWS, dd), lambda i: (i, 0))
  blk_part = lambda dd: pl.BlockSpec((NC, _ROWS, dd), lambda i: (0, i, 0))
  blk_w = pl.BlockSpec((d_in, d_hid), lambda i: (0, 0))
  blk_b = lambda dd: pl.BlockSpec((1, dd), lambda i: (0, 0))

  h1p = pl.pallas_call(
      _tc_matmul1,
      grid=grid,
      in_specs=[blk_rows(d_in), blk_w, blk_part(LANES)],
      out_specs=blk_rows(d_hid),
      out_shape=jax.ShapeDtypeStruct((n, d_hid), jnp.float32),
  )(node_embed, W1, degp)

  scat = _edge_scatter(n, n_chunks, d_hid)
  part1 = scat(h1p, src, dst)

  h2p = pl.pallas_call(
      _tc_mid,
      grid=grid,
      in_specs=[blk_part(d_hid), blk_rows(d_hid), blk_part(LANES),
                pl.BlockSpec((d_hid, d_out), lambda i: (0, 0)), blk_b(d_hid)],
      out_specs=blk_rows(d_out),
      out_shape=jax.ShapeDtypeStruct((n, d_out), jnp.float32),
  )(part1, h1p, degp, W2, b1r)

  part2 = scat(h2p, src, dst)

  out = pl.pallas_call(
      _tc_final,
      grid=grid,
      in_specs=[blk_part(d_out), blk_rows(d_out), blk_part(LANES),
                blk_b(d_out)],
      out_specs=blk_rows(d_out),
      out_shape=jax.ShapeDtypeStruct((n, d_out), jnp.float32),
  )(part2, h2p, degp, b2r)

  return out


# speedup vs baseline: 34.8614x; 1.0141x over previous
import functools

import jax
import jax.numpy as jnp
from jax import lax
from jax.experimental import pallas as pl
from jax.experimental.pallas import tpu as pltpu
from jax.experimental.pallas import tpu_sc as plsc

NC = 2
NS = 16
NW = NC * NS
LANES = 16
CHUNK = 80
RPT = 632

_MESH = plsc.VectorSubcoreMesh(
    core_axis_name="c", subcore_axis_name="s", num_cores=NC, num_subcores=NS
)


def _deg_partials(n_nodes, n_chunks):
  last_rows = n_nodes - (NS - 1) * RPT
  ring = 8

  @functools.partial(
      pl.kernel,
      out_type=jax.ShapeDtypeStruct((NC, n_nodes, LANES), jnp.float32),
      mesh=_MESH,
      scratch_types=[
          pltpu.VMEM_SHARED((n_nodes, LANES), jnp.float32),
          pltpu.VMEM((RPT, LANES), jnp.float32),
          pltpu.VMEM((CHUNK, LANES), jnp.float32),
          pltpu.VMEM((n_chunks, CHUNK), jnp.int32),
          pltpu.SemaphoreType.DMA,
      ],
  )
  def k(dst_hbm, degp_hbm, acc_sh, zeros_v, ones_v, dsts_v, ssem):
    cid = lax.axis_index("c")
    sid = lax.axis_index("s")
    wid = cid * NS + sid

    @pl.loop(0, RPT)
    def _(r):
      zeros_v[r, :] = jnp.zeros((LANES,), jnp.float32)

    @pl.loop(0, CHUNK)
    def _(r):
      ones_v[r, :] = jnp.full((LANES,), 1.0, jnp.float32)

    pltpu.sync_copy(dst_hbm.at[wid], dsts_v)

    @pl.when(sid < NS - 1)
    def _():
      pltpu.sync_copy(zeros_v, acc_sh.at[pl.ds(sid * RPT, RPT)])

    @pl.when(sid == NS - 1)
    def _():
      pltpu.sync_copy(zeros_v.at[pl.ds(0, last_rows)],
                      acc_sh.at[pl.ds((NS - 1) * RPT, last_rows)])

    plsc.subcore_barrier()

    def drain():
      pltpu.make_async_copy(ones_v, acc_sh.at[dsts_v.at[0]], ssem).wait()

    @pl.loop(0, n_chunks)
    def _(i):
      @pl.when(i >= ring)
      def _():
        drain()
      pltpu.async_copy(ones_v, acc_sh.at[dsts_v.at[i]], ssem, add=True)

    for _ in range(min(ring, n_chunks)):
      drain()

    plsc.subcore_barrier()

    @pl.when(sid < NS - 1)
    def _():
      pltpu.sync_copy(acc_sh.at[pl.ds(sid * RPT, RPT)],
                      degp_hbm.at[cid, pl.ds(sid * RPT, RPT)])

    @pl.when(sid == NS - 1)
    def _():
      pltpu.sync_copy(acc_sh.at[pl.ds((NS - 1) * RPT, last_rows)],
                      degp_hbm.at[cid, pl.ds((NS - 1) * RPT, last_rows)])

  return k


def _edge_scatter(n_nodes, n_chunks, d):
  last_rows = n_nodes - (NS - 1) * RPT
  assert n_chunks >= 13

  @functools.partial(
      pl.kernel,
      out_type=jax.ShapeDtypeStruct((NC, n_nodes, d), jnp.float32),
      mesh=_MESH,
      scratch_types=[
          pltpu.VMEM_SHARED((n_nodes, d), jnp.float32),
          [pltpu.VMEM((CHUNK, d), jnp.float32)] * 4,
          [pltpu.VMEM((CHUNK,), jnp.int32)] * 8,
          [pltpu.VMEM((CHUNK,), jnp.int32)] * 8,
          [pltpu.SemaphoreType.DMA] * 4,
          [pltpu.SemaphoreType.DMA] * 4,
          [pltpu.SemaphoreType.DMA] * 8,
          pltpu.SemaphoreType.DMA,
      ],
  )
  def k(h_hbm, src_hbm, dst_hbm, part_hbm,
        acc_sh, rows, sb, db, gsem, ssem, isem, psem):
    cid = lax.axis_index("c")
    sid = lax.axis_index("s")
    wid = cid * NS + sid
    base = wid * (n_chunks * CHUNK)

    @pl.when(sid < NS - 1)
    def _():
      pltpu.async_copy(h_hbm.at[pl.ds(sid * RPT, RPT)],
                       acc_sh.at[pl.ds(sid * RPT, RPT)], psem)

    @pl.when(sid == NS - 1)
    def _():
      pltpu.async_copy(h_hbm.at[pl.ds((NS - 1) * RPT, last_rows)],
                       acc_sh.at[pl.ds((NS - 1) * RPT, last_rows)], psem)

    def li(j, q):
      off = base + j * CHUNK
      pltpu.async_copy(src_hbm.at[pl.ds(off, CHUNK)], sb[q], isem[q])
      pltpu.async_copy(dst_hbm.at[pl.ds(off, CHUNK)], db[q], isem[q])

    def wi(q):
      pltpu.make_async_copy(src_hbm.at[pl.ds(0, CHUNK)], sb[q], isem[q]).wait()
      pltpu.make_async_copy(dst_hbm.at[pl.ds(0, CHUNK)], db[q], isem[q]).wait()

    def g(b, q):
      pltpu.async_copy(h_hbm.at[sb[q]], rows[b], gsem[b])

    def wg(b):
      pltpu.make_async_copy(h_hbm.at[sb[0]], rows[b], gsem[b]).wait()

    def s(b, q):
      pltpu.async_copy(rows[b], acc_sh.at[db[q]], ssem[b], add=True)

    def ws(b):
      pltpu.make_async_copy(rows[0], acc_sh.at[db[0]], ssem[b]).wait()

    for q in range(8):
      li(q, q)
    wi(0)
    g(0, 0)
    wi(1)
    g(1, 1)

    @pl.when(sid < NS - 1)
    def _():
      pltpu.make_async_copy(h_hbm.at[pl.ds(0, RPT)],
                            acc_sh.at[pl.ds(0, RPT)], psem).wait()

    @pl.when(sid == NS - 1)
    def _():
      pltpu.make_async_copy(h_hbm.at[pl.ds(0, last_rows)],
                            acc_sh.at[pl.ds(0, last_rows)], psem).wait()

    plsc.subcore_barrier()

    wg(0)
    s(0, 0)
    wi(2)
    g(2, 2)
    wg(1)
    s(1, 1)
    wi(3)
    g(3, 3)

    n_oct = (n_chunks - 8) // 8

    @pl.loop(0, n_oct)
    def _(i):
      j0 = 8 * i + 4
      for t in range(8):
        b, q = t % 4, (4 + t) % 8
        ws(b)
        li(j0 + t + 4, t)
        wi(q)
        g(b, q)
        wg((t + 2) % 4)
        s((t + 2) % 4, (2 + t) % 8)

    for j in range(8 * n_oct + 4, n_chunks):
      b, q = j % 4, j % 8
      ws(b)
      if j + 4 < n_chunks:
        li(j + 4, (j + 4) % 8)
      wi(q)
      g(b, q)
      wg((j - 2) % 4)
      s((j - 2) % 4, (j - 2) % 8)
    for j in (n_chunks - 2, n_chunks - 1):
      wg(j % 4)
      s(j % 4, j % 8)
    for b in range(4):
      ws(b)

    plsc.subcore_barrier()

    @pl.when(sid < NS - 1)
    def _():
      pltpu.sync_copy(acc_sh.at[pl.ds(sid * RPT, RPT)],
                      part_hbm.at[cid, pl.ds(sid * RPT, RPT)])

    @pl.when(sid == NS - 1)
    def _():
      pltpu.sync_copy(acc_sh.at[pl.ds((NS - 1) * RPT, last_rows)],
                      part_hbm.at[cid, pl.ds((NS - 1) * RPT, last_rows)])

  return k


_ROWS = 2000


def _dis_from(degp_ref):
  deg = degp_ref[0, :, 0:1] + degp_ref[1, :, 0:1] + 1.0
  return lax.rsqrt(deg)


def _tc_matmul1(x_ref, w_ref, degp_ref, o_ref):
  dis = _dis_from(degp_ref)
  h = jnp.dot(x_ref[...], w_ref[...], preferred_element_type=jnp.float32)
  o_ref[...] = h * dis


def _tc_mid(part_ref, h_ref, degp_ref, w_ref, b_ref, o_ref):
  dis = _dis_from(degp_ref)
  s = part_ref[0] + part_ref[1] - h_ref[...]
  z = jnp.maximum(dis * s + b_ref[...], 0.0)
  h2 = jnp.dot(z, w_ref[...], preferred_element_type=jnp.float32)
  o_ref[...] = h2 * dis


def _tc_final(part_ref, h_ref, degp_ref, b_ref, o_ref):
  dis = _dis_from(degp_ref)
  s = part_ref[0] + part_ref[1] - h_ref[...]
  o_ref[...] = dis * s + b_ref[...]


def kernel(node_embed, message_edge, W1, b1, W2, b2):
  n, d_in = node_embed.shape
  d_hid = W1.shape[1]
  d_out = W2.shape[1]
  e = message_edge.shape[1]
  ept = e // NW
  n_chunks = ept // CHUNK
  assert ept % CHUNK == 0 and e % NW == 0
  src = message_edge[0]
  dst = message_edge[1]
  dst3 = dst.reshape(NW, n_chunks, CHUNK)
  b1r = b1.reshape(1, d_hid)
  b2r = b2.reshape(1, d_out)

  degp = _deg_partials(n, n_chunks)(dst3)

  grid = (n // _ROWS,)
  blk_rows = lambda dd: pl.BlockSpec((_ROWS, dd), lambda i: (i, 0))
  blk_part = lambda dd: pl.BlockSpec((NC, _ROWS, dd), lambda i: (0, i, 0))
  blk_w = pl.BlockSpec((d_in, d_hid), lambda i: (0, 0))
  blk_b = lambda dd: pl.BlockSpec((1, dd), lambda i: (0, 0))

  h1p = pl.pallas_call(
      _tc_matmul1,
      grid=grid,
      in_specs=[blk_rows(d_in), blk_w, blk_part(LANES)],
      out_specs=blk_rows(d_hid),
      out_shape=jax.ShapeDtypeStruct((n, d_hid), jnp.float32),
  )(node_embed, W1, degp)

  scat = _edge_scatter(n, n_chunks, d_hid)
  part1 = scat(h1p, src, dst)

  h2p = pl.pallas_call(
      _tc_mid,
      grid=grid,
      in_specs=[blk_part(d_hid), blk_rows(d_hid), blk_part(LANES),
                pl.BlockSpec((d_hid, d_out), lambda i: (0, 0)), blk_b(d_hid)],
      out_specs=blk_rows(d_out),
      out_shape=jax.ShapeDtypeStruct((n, d_out), jnp.float32),
  )(part1, h1p, degp, W2, b1r)

  part2 = scat(h2p, src, dst)

  out = pl.pallas_call(
      _tc_final,
      grid=grid,
      in_specs=[blk_part(d_out), blk_rows(d_out), blk_part(LANES),
                blk_b(d_out)],
      out_specs=blk_rows(d_out),
      out_shape=jax.ShapeDtypeStruct((n, d_out), jnp.float32),
  )(part2, h2p, degp, b2r)

  return out
